# baseline (speedup 1.0000x reference)
import functools

import jax
import jax.numpy as jnp
from jax import lax
from jax.experimental import pallas as pl
from jax.experimental.pallas import tpu as pltpu
from jax.experimental.pallas import tpu_sc as plsc

VOCAB = 100000
MAXLEN = 200
EMBED = 128
BATCH = 4096

NC = 2
NS = 16
NW = NC * NS
LANES = 16

ROWS = BATCH * MAXLEN
CHUNK = 80
NCHUNK = ROWS // CHUNK
CPW = NCHUNK // NW
NB = 5


def _body(idx_hbm, tok_hbm, pos_hbm, out_hbm, pos_v, idx_v, buf, isem, gsem, ssem):
    wid = lax.axis_index("s") * NC + lax.axis_index("c")
    cbase = wid * CPW

    def idx_copy(b, cid):
        return pltpu.make_async_copy(idx_hbm.at[cid], idx_v.at[b], isem.at[b])

    def gather(b):
        return pltpu.make_async_copy(tok_hbm.at[idx_v.at[b]], buf.at[b], gsem.at[b])

    def scatter(b, cid):
        return pltpu.make_async_copy(
            buf.at[b], out_hbm.at[pl.ds(cid * CHUNK, CHUNK)], ssem.at[b])

    pltpu.sync_copy(pos_hbm, pos_v.at[pl.ds(0, MAXLEN)])
    pltpu.sync_copy(pos_hbm.at[pl.ds(0, CHUNK)], pos_v.at[pl.ds(MAXLEN, CHUNK)])

    for b in range(NB):
        idx_copy(b, cbase + b).start()
    for b in range(2):
        idx_copy(b, cbase + b).wait()
        gather(b).start()

    @pl.loop(0, CPW, step=NB)
    def _ring(g):
        for b in range(NB):
            t = g + b
            cid = cbase + t
            gather(b).wait()

            @pl.when(t + NB < CPW)
            def _():
                idx_copy(b, cbase + t + NB).start()

            poff = (b * CHUNK) % MAXLEN

            @pl.loop(0, CHUNK)
            def _rows(m):
                for c in range(EMBED // LANES):
                    sl = pl.ds(c * LANES, LANES)
                    plsc.addupdate(buf.at[b, m, sl], pos_v[poff + m, sl])

            scatter(b, cid).start()

            b2 = (b + 2) % NB
            @pl.when(t + 2 < CPW)
            def _():
                @pl.when(t >= NB - 2)
                def _():
                    scatter(b2, cbase).wait()
                idx_copy(b2, cbase).wait()
                gather(b2).start()

    for b in range(NB):
        scatter(b, cbase).wait()


@functools.partial(jax.jit, static_argnums=())
def _run(idx, token_table, pos_table):
    kern = pl.kernel(
        _body,
        out_type=jax.ShapeDtypeStruct((ROWS, EMBED), jnp.float32),
        mesh=plsc.VectorSubcoreMesh(core_axis_name="c", subcore_axis_name="s"),
        scratch_types=[
            pltpu.VMEM((MAXLEN + CHUNK, EMBED), jnp.float32),
            pltpu.VMEM((NB, CHUNK), jnp.int32),
            pltpu.VMEM((NB, CHUNK, EMBED), jnp.float32),
            pltpu.SemaphoreType.DMA((NB,)),
            pltpu.SemaphoreType.DMA((NB,)),
            pltpu.SemaphoreType.DMA((NB,)),
        ],
    )
    return kern(idx, token_table, pos_table)


def kernel(inputs, token_table, pos_table):
    idx = inputs.astype(jnp.int32).reshape(NCHUNK, CHUNK)
    out = _run(idx, token_table, pos_table)
    return out.reshape(BATCH, MAXLEN, EMBED)


# speedup vs baseline: 1.1225x; 1.1225x over previous
import functools

import jax
import jax.numpy as jnp
from jax import lax
from jax.experimental import pallas as pl
from jax.experimental.pallas import tpu as pltpu
from jax.experimental.pallas import tpu_sc as plsc

VOCAB = 100000
MAXLEN = 200
EMBED = 128
BATCH = 4096

NC = 2
NS = 16
NW = NC * NS
LANES = 16

ROWS = BATCH * MAXLEN
CHUNK = 40
NCHUNK = ROWS // CHUNK
CPW = NCHUNK // NW
NB = 10
LG = 4


def _body(idx_hbm, tok_hbm, pos_hbm, out_hbm, pos_v, idx_v, buf, isem, gsem, ssem):
    wid = lax.axis_index("s") * NC + lax.axis_index("c")
    cbase = wid * CPW

    def idx_copy(b, cid):
        return pltpu.make_async_copy(idx_hbm.at[cid], idx_v.at[b], isem.at[b])

    def gather(b):
        return pltpu.make_async_copy(tok_hbm.at[idx_v.at[b]], buf.at[b], gsem.at[b])

    def scatter(b, cid):
        return pltpu.make_async_copy(
            buf.at[b], out_hbm.at[pl.ds(cid * CHUNK, CHUNK)], ssem.at[b])

    pltpu.sync_copy(pos_hbm, pos_v.at[pl.ds(0, MAXLEN)])
    pltpu.sync_copy(pos_hbm.at[pl.ds(0, CHUNK)], pos_v.at[pl.ds(MAXLEN, CHUNK)])

    for b in range(NB):
        idx_copy(b, cbase + b).start()
    for b in range(LG):
        idx_copy(b, cbase + b).wait()
        gather(b).start()

    @pl.loop(0, CPW, step=NB)
    def _ring(g):
        for b in range(NB):
            t = g + b
            cid = cbase + t
            gather(b).wait()

            @pl.when(t + NB < CPW)
            def _():
                idx_copy(b, cbase + t + NB).start()

            poff = (b * CHUNK) % MAXLEN

            @pl.loop(0, CHUNK)
            def _rows(m):
                for c in range(EMBED // LANES):
                    sl = pl.ds(c * LANES, LANES)
                    plsc.addupdate(buf.at[b, m, sl], pos_v[poff + m, sl])

            scatter(b, cid).start()

            b2 = (b + LG) % NB
            @pl.when(t + LG < CPW)
            def _():
                @pl.when(t >= NB - LG)
                def _():
                    scatter(b2, cbase).wait()
                idx_copy(b2, cbase).wait()
                gather(b2).start()

    for b in range(NB):
        scatter(b, cbase).wait()


@functools.partial(jax.jit, static_argnums=())
def _run(idx, token_table, pos_table):
    kern = pl.kernel(
        _body,
        out_type=jax.ShapeDtypeStruct((ROWS, EMBED), jnp.float32),
        mesh=plsc.VectorSubcoreMesh(core_axis_name="c", subcore_axis_name="s"),
        scratch_types=[
            pltpu.VMEM((MAXLEN + CHUNK, EMBED), jnp.float32),
            pltpu.VMEM((NB, CHUNK), jnp.int32),
            pltpu.VMEM((NB, CHUNK, EMBED), jnp.float32),
            pltpu.SemaphoreType.DMA((NB,)),
            pltpu.SemaphoreType.DMA((NB,)),
            pltpu.SemaphoreType.DMA((NB,)),
        ],
    )
    return kern(idx, token_table, pos_table)


def kernel(inputs, token_table, pos_table):
    idx = inputs.astype(jnp.int32).reshape(NCHUNK, CHUNK)
    out = _run(idx, token_table, pos_table)
    return out.reshape(BATCH, MAXLEN, EMBED)


# speedup vs baseline: 1.1709x; 1.0431x over previous
import functools

import jax
import jax.numpy as jnp
from jax import lax
from jax.experimental import pallas as pl
from jax.experimental.pallas import tpu as pltpu
from jax.experimental.pallas import tpu_sc as plsc

VOCAB = 100000
MAXLEN = 200
EMBED = 128
BATCH = 4096

NC = 2
NS = 16
NW = NC * NS
LANES = 16

ROWS = BATCH * MAXLEN
CHUNK = 40
NCHUNK = ROWS // CHUNK
CPW = NCHUNK // NW
NB = 10
LG = 5


def _body(idx_hbm, tok_hbm, pos_hbm, out_hbm, pos_v, idx_v, buf, isem, gsem, ssem):
    wid = lax.axis_index("s") * NC + lax.axis_index("c")
    cbase = wid * CPW

    def idx_copy(b, cid):
        return pltpu.make_async_copy(idx_hbm.at[cid], idx_v.at[b], isem.at[b])

    def gather(b):
        return pltpu.make_async_copy(tok_hbm.at[idx_v.at[b]], buf.at[b], gsem.at[b])

    def scatter(b, cid):
        return pltpu.make_async_copy(
            buf.at[b], out_hbm.at[pl.ds(cid * CHUNK, CHUNK)], ssem.at[b])

    pltpu.sync_copy(pos_hbm, pos_v.at[pl.ds(0, MAXLEN)])
    pltpu.sync_copy(pos_hbm.at[pl.ds(0, CHUNK)], pos_v.at[pl.ds(MAXLEN, CHUNK)])

    for b in range(NB):
        idx_copy(b, cbase + b).start()
    for b in range(LG):
        idx_copy(b, cbase + b).wait()
        gather(b).start()

    @pl.loop(0, CPW, step=NB)
    def _ring(g):
        for b in range(NB):
            t = g + b
            cid = cbase + t
            gather(b).wait()

            @pl.when(t + NB < CPW)
            def _():
                idx_copy(b, cbase + t + NB).start()

            poff = (b * CHUNK) % MAXLEN

            @pl.loop(0, CHUNK)
            def _rows(m):
                for c in range(EMBED // LANES):
                    sl = pl.ds(c * LANES, LANES)
                    plsc.addupdate(buf.at[b, m, sl], pos_v[poff + m, sl])

            scatter(b, cid).start()

            b2 = (b + LG) % NB
            @pl.when(t + LG < CPW)
            def _():
                @pl.when(t >= NB - LG)
                def _():
                    scatter(b2, cbase).wait()
                idx_copy(b2, cbase).wait()
                gather(b2).start()

    for b in range(NB):
        scatter(b, cbase).wait()


@functools.partial(jax.jit, static_argnums=())
def _run(idx, token_table, pos_table):
    kern = pl.kernel(
        _body,
        out_type=jax.ShapeDtypeStruct((ROWS, EMBED), jnp.float32),
        mesh=plsc.VectorSubcoreMesh(core_axis_name="c", subcore_axis_name="s"),
        scratch_types=[
            pltpu.VMEM((MAXLEN + CHUNK, EMBED), jnp.float32),
            pltpu.VMEM((NB, CHUNK), jnp.int32),
            pltpu.VMEM((NB, CHUNK, EMBED), jnp.float32),
            pltpu.SemaphoreType.DMA((NB,)),
            pltpu.SemaphoreType.DMA((NB,)),
            pltpu.SemaphoreType.DMA((NB,)),
        ],
    )
    return kern(idx, token_table, pos_table)


def kernel(inputs, token_table, pos_table):
    idx = inputs.astype(jnp.int32).reshape(NCHUNK, CHUNK)
    out = _run(idx, token_table, pos_table)
    return out.reshape(BATCH, MAXLEN, EMBED)


# speedup vs baseline: 1.1810x; 1.0086x over previous
import functools

import jax
import jax.numpy as jnp
from jax import lax
from jax.experimental import pallas as pl
from jax.experimental.pallas import tpu as pltpu
from jax.experimental.pallas import tpu_sc as plsc

VOCAB = 100000
MAXLEN = 200
EMBED = 128
BATCH = 4096

NC = 2
NS = 16
NW = NC * NS
LANES = 16

ROWS = BATCH * MAXLEN
CHUNK = 40
NCHUNK = ROWS // CHUNK
CPW = NCHUNK // NW
NB = 10
LG = 6


def _body(idx_hbm, tok_hbm, pos_hbm, out_hbm, pos_v, idx_v, buf, isem, gsem, ssem):
    wid = lax.axis_index("s") * NC + lax.axis_index("c")
    cbase = wid * CPW

    def idx_copy(b, cid):
        return pltpu.make_async_copy(idx_hbm.at[cid], idx_v.at[b], isem.at[b])

    def gather(b):
        return pltpu.make_async_copy(tok_hbm.at[idx_v.at[b]], buf.at[b], gsem.at[b])

    def scatter(b, cid):
        return pltpu.make_async_copy(
            buf.at[b], out_hbm.at[pl.ds(cid * CHUNK, CHUNK)], ssem.at[b])

    pltpu.sync_copy(pos_hbm, pos_v.at[pl.ds(0, MAXLEN)])
    pltpu.sync_copy(pos_hbm.at[pl.ds(0, CHUNK)], pos_v.at[pl.ds(MAXLEN, CHUNK)])

    for b in range(NB):
        idx_copy(b, cbase + b).start()
    for b in range(LG):
        idx_copy(b, cbase + b).wait()
        gather(b).start()

    @pl.loop(0, CPW, step=NB)
    def _ring(g):
        for b in range(NB):
            t = g + b
            cid = cbase + t
            gather(b).wait()

            @pl.when(t + NB < CPW)
            def _():
                idx_copy(b, cbase + t + NB).start()

            poff = (b * CHUNK) % MAXLEN

            @pl.loop(0, CHUNK)
            def _rows(m):
                for c in range(EMBED // LANES):
                    sl = pl.ds(c * LANES, LANES)
                    plsc.addupdate(buf.at[b, m, sl], pos_v[poff + m, sl])

            scatter(b, cid).start()

            b2 = (b + LG) % NB
            @pl.when(t + LG < CPW)
            def _():
                @pl.when(t >= NB - LG)
                def _():
                    scatter(b2, cbase).wait()
                idx_copy(b2, cbase).wait()
                gather(b2).start()

    for b in range(NB):
        scatter(b, cbase).wait()


@functools.partial(jax.jit, static_argnums=())
def _run(idx, token_table, pos_table):
    kern = pl.kernel(
        _body,
        out_type=jax.ShapeDtypeStruct((ROWS, EMBED), jnp.float32),
        mesh=plsc.VectorSubcoreMesh(core_axis_name="c", subcore_axis_name="s"),
        scratch_types=[
            pltpu.VMEM((MAXLEN + CHUNK, EMBED), jnp.float32),
            pltpu.VMEM((NB, CHUNK), jnp.int32),
            pltpu.VMEM((NB, CHUNK, EMBED), jnp.float32),
            pltpu.SemaphoreType.DMA((NB,)),
            pltpu.SemaphoreType.DMA((NB,)),
            pltpu.SemaphoreType.DMA((NB,)),
        ],
    )
    return kern(idx, token_table, pos_table)


def kernel(inputs, token_table, pos_table):
    idx = inputs.astype(jnp.int32).reshape(NCHUNK, CHUNK)
    out = _run(idx, token_table, pos_table)
    return out.reshape(BATCH, MAXLEN, EMBED)
